# baseline (speedup 1.0000x reference)
import jax
import jax.numpy as jnp
from jax.experimental import pallas as pl
from jax.experimental.pallas import tpu as pltpu

B = 16
D = 8732
C = 21
TOPK = 200
CONF_THRESH = 0.01
NMS_THRESH = 0.45

DP = 8832
SUB = DP // 128
PAD = DP - D
NSLOT = 256
BIG = 2 ** 30


def _softmax_kernel(x_ref, o_ref):
    x = x_ref[...]
    m = jnp.max(x, axis=-1, keepdims=True)
    e = jnp.exp(x - m)
    o_ref[...] = e / jnp.sum(e, axis=-1, keepdims=True)


def _softmax(x):
    return pl.pallas_call(
        _softmax_kernel,
        out_shape=jax.ShapeDtypeStruct(x.shape, x.dtype),
        grid=(B,),
        in_specs=[pl.BlockSpec((1, D, C), lambda b: (b, 0, 0))],
        out_specs=pl.BlockSpec((1, D, C), lambda b: (b, 0, 0)),
    )(x)


def _decode(lref, db_ref):
    lcx = lref[0, 0]
    lcy = lref[0, 1]
    lw = lref[0, 2]
    lh = lref[0, 3]
    dcx = db_ref[0]
    dcy = db_ref[1]
    dw = db_ref[2]
    dh = db_ref[3]
    cx = dcx + lcx * 0.1 * dcx
    cy = dcy + lcy * 0.1 * dcy
    w = dw * jnp.exp(lw * 0.2)
    h = dh * jnp.exp(lh * 0.2)
    x1 = cx - w * 0.5
    y1 = cy - h * 0.5
    return x1, y1, x1 + w, y1 + h


def _nms_kernel(sa_ref, sb_ref, la_ref, lb_ref, db_ref, o_ref, iou_ref):
    f32 = jnp.float32
    sa = sa_ref[0, 0]
    sb = sb_ref[0, 0]
    sA = jnp.where(sa > CONF_THRESH, sa, 0.0)
    sB = jnp.where(sb > CONF_THRESH, sb, 0.0)

    x1a, y1a, x2a, y2a = _decode(la_ref, db_ref)
    fx1, fy1, fx2, fy2 = _decode(lb_ref, db_ref)
    x1b = 1.0 - fx2
    y1b = fy1
    x2b = 1.0 - fx1
    y2b = fy2

    idxA = (jax.lax.broadcasted_iota(jnp.int32, (SUB, 128), 0) * 128
            + jax.lax.broadcasted_iota(jnp.int32, (SUB, 128), 1))
    idxB = idxA + DP
    lane_r = jax.lax.broadcasted_iota(jnp.int32, (1, NSLOT), 1)
    row_c = jax.lax.broadcasted_iota(jnp.int32, (NSLOT, 1), 0)

    zr = jnp.zeros((1, NSLOT), f32)
    zc = jnp.zeros((NSLOT, 1), f32)

    def topk_body(i, carry):
        sA, sB, ts, x1c, x1r, y1c, y1r, x2c, x2r, y2c, y2r = carry
        m = jnp.maximum(jnp.max(sA), jnp.max(sB))
        candA = jnp.where(sA == m, idxA, BIG)
        candB = jnp.where(sB == m, idxB, BIG)
        amin = jnp.minimum(jnp.min(candA), jnp.min(candB))
        ohA = idxA == amin
        ohB = idxB == amin

        def ext(pa, pb):
            return (jnp.sum(jnp.where(ohA, pa, 0.0))
                    + jnp.sum(jnp.where(ohB, pb, 0.0)))

        vx1 = ext(x1a, x1b)
        vy1 = ext(y1a, y1b)
        vx2 = ext(x2a, x2b)
        vy2 = ext(y2a, y2b)
        selr = lane_r == i
        selc = row_c == i
        ts = jnp.where(selr, m, ts)
        x1c = jnp.where(selc, vx1, x1c)
        x1r = jnp.where(selr, vx1, x1r)
        y1c = jnp.where(selc, vy1, y1c)
        y1r = jnp.where(selr, vy1, y1r)
        x2c = jnp.where(selc, vx2, x2c)
        x2r = jnp.where(selr, vx2, x2r)
        y2c = jnp.where(selc, vy2, y2c)
        y2r = jnp.where(selr, vy2, y2r)
        sA = jnp.where(ohA, -1.0, sA)
        sB = jnp.where(ohB, -1.0, sB)
        return sA, sB, ts, x1c, x1r, y1c, y1r, x2c, x2r, y2c, y2r

    init = (sA, sB, zr, zc, zr, zc, zr, zc, zr, zc, zr)
    (_, _, ts, x1c, x1r, y1c, y1r, x2c, x2r, y2c, y2r) = jax.lax.fori_loop(
        0, TOPK, topk_body, init)

    area_c = (x2c - x1c) * (y2c - y1c)
    area_r = (x2r - x1r) * (y2r - y1r)
    xx1 = jnp.maximum(x1c, x1r)
    yy1 = jnp.maximum(y1c, y1r)
    xx2 = jnp.minimum(x2c, x2r)
    yy2 = jnp.minimum(y2c, y2r)
    w = jnp.clip(xx2 - xx1, 0.0, None)
    h = jnp.clip(yy2 - yy1, 0.0, None)
    inter = w * h
    union = area_c + area_r - inter
    denom = jnp.where(union <= 0.0, 1.0, union)
    iou_ref[...] = jnp.where(union <= 0.0, 0.0, inter / denom)

    valid_r = (ts > CONF_THRESH).astype(f32)

    def scan_body(i, carry):
        keep_r, keep_c = carry
        row = iou_ref[pl.ds(i, 1), :]
        earlier = keep_r * (lane_r < i).astype(f32)
        sup = jnp.max(earlier * (row > NMS_THRESH).astype(f32))
        validi = jnp.max(valid_r * (lane_r == i).astype(f32))
        newv = jnp.where(sup > 0.0, 0.0, validi)
        keep_r = jnp.where(lane_r == i, newv, keep_r)
        keep_c = jnp.where(row_c == i, newv, keep_c)
        return keep_r, keep_c

    keep_r, keep_c = jax.lax.fori_loop(0, TOPK, scan_body, (zr, zc))

    colio = jax.lax.broadcasted_iota(jnp.int32, (NSLOT, NSLOT), 1)
    rowio = jax.lax.broadcasted_iota(jnp.int32, (NSLOT, NSLOT), 0)
    csum_c = jnp.sum((colio <= rowio).astype(f32) * keep_r, axis=1,
                     keepdims=True)
    rank_c = csum_c - 1.0
    perm = ((colio.astype(f32) == rank_c) & (keep_c > 0.0)).astype(f32)
    vals = jnp.concatenate(
        [ts, x1r, y1r, x2r, y2r, jnp.zeros((3, NSLOT), f32)], axis=0)
    packed = jnp.dot(vals, perm, preferred_element_type=f32)
    cls = pl.program_id(1)
    packed = packed * jnp.where(cls == 0, 0.0, 1.0)
    o_ref[0, 0] = packed


def _pad_last(x):
    pads = [(0, 0)] * (x.ndim - 1) + [(0, PAD)]
    return jnp.pad(x, pads)


def kernel(loc_data, conf_data, loc_data2, conf_data2, dbox_list):
    cp = _softmax(conf_data)
    cp2 = _softmax(conf_data2)
    sa = _pad_last(cp.transpose(0, 2, 1)).reshape(B, C, SUB, 128)
    sb = _pad_last(cp2.transpose(0, 2, 1)).reshape(B, C, SUB, 128)
    la = _pad_last(loc_data.transpose(0, 2, 1)).reshape(B, 4, SUB, 128)
    lb = _pad_last(loc_data2.transpose(0, 2, 1)).reshape(B, 4, SUB, 128)
    db = _pad_last(dbox_list.T).reshape(4, SUB, 128)

    out = pl.pallas_call(
        _nms_kernel,
        grid=(B, C),
        in_specs=[
            pl.BlockSpec((1, 1, SUB, 128), lambda b, c: (b, c, 0, 0)),
            pl.BlockSpec((1, 1, SUB, 128), lambda b, c: (b, c, 0, 0)),
            pl.BlockSpec((1, 4, SUB, 128), lambda b, c: (b, 0, 0, 0)),
            pl.BlockSpec((1, 4, SUB, 128), lambda b, c: (b, 0, 0, 0)),
            pl.BlockSpec((4, SUB, 128), lambda b, c: (0, 0, 0)),
        ],
        out_specs=pl.BlockSpec((1, 1, 8, NSLOT), lambda b, c: (b, c, 0, 0)),
        out_shape=jax.ShapeDtypeStruct((B, C, 8, NSLOT), jnp.float32),
        scratch_shapes=[pltpu.VMEM((NSLOT, NSLOT), jnp.float32)],
        compiler_params=pltpu.CompilerParams(
            dimension_semantics=("parallel", "parallel")),
    )(sa, sb, la, lb, db)

    return out.transpose(0, 1, 3, 2)[:, :, :TOPK, :5]
